# Initial kernel scaffold; baseline (speedup 1.0000x reference)
import jax
import jax.numpy as jnp
from jax.experimental import pallas as pl


def kernel(x, edge_index, W1, b1, W2, b2, W3, b3, Ws1, Wn1, bS1, Wng, bg, Ws3, Wn3, bS3, Wc, bc):
    raise NotImplementedError("write your pallas kernel here")


# speedup vs baseline: 2.6760x; 2.6760x over previous
import functools

import jax
import jax.numpy as jnp
from jax import lax
from jax.experimental import pallas as pl
from jax.experimental.pallas import tpu as pltpu
from jax.experimental.pallas import tpu_sc as plsc

N = 10000
NPAD = 10240
E = 160000
LANE = 128
CHUNK = 80
NSUB = 16
EDGES_PER_SUB = E // NSUB
NCHUNKS = EDGES_PER_SUB // CHUNK
ROWS_PER_SUB = NPAD // NSUB
ZROWS = 128
DEGW = 128
R = 1000
NBLK = N // R


def _make_agg(nsl):
    S = nsl // 2
    mesh = plsc.VectorSubcoreMesh(core_axis_name="c", subcore_axis_name="s")
    out_type = [jax.ShapeDtypeStruct((NPAD, LANE), jnp.float32)
                for _ in range(nsl)]
    scratch_types = [
        pltpu.VMEM((CHUNK,), jnp.int32),
        pltpu.VMEM((CHUNK,), jnp.int32),
        pltpu.VMEM((CHUNK, LANE), jnp.float32),
        pltpu.VMEM((ZROWS, LANE), jnp.float32),
        pltpu.VMEM_SHARED((NPAD, LANE), jnp.float32),
        pltpu.SemaphoreType.DMA,
    ]

    @functools.partial(pl.kernel, mesh=mesh, out_type=out_type,
                       scratch_types=scratch_types)
    def agg(*args):
        h_refs = args[:nsl]
        src_hbm = args[nsl]
        dst_hbm = args[nsl + 1]
        outs = args[nsl + 2:nsl + 2 + nsl]
        idx_s, idx_d, buf, zbuf, acc, sem = args[nsl + 2 + nsl:]
        c = lax.axis_index("c")
        s = lax.axis_index("s")
        row0 = s * ROWS_PER_SUB

        zero16 = jnp.zeros((16,), jnp.float32)

        def zrow(i, _):
            for j in range(LANE // 16):
                zbuf[i, pl.ds(j * 16, 16)] = zero16
            return 0
        lax.fori_loop(0, ZROWS, zrow, 0)

        for sl in range(S):
            for z in range(ROWS_PER_SUB // ZROWS):
                pltpu.sync_copy(zbuf, acc.at[pl.ds(row0 + z * ZROWS, ZROWS)])
            plsc.subcore_barrier()

            for core in range(2):
                @pl.when(c == core)
                def _(table=h_refs[core * S + sl]):
                    def chunk(i, _):
                        off = pl.multiple_of(
                            s * EDGES_PER_SUB + i * CHUNK, CHUNK)
                        pltpu.sync_copy(src_hbm.at[pl.ds(off, CHUNK)], idx_s)
                        pltpu.sync_copy(dst_hbm.at[pl.ds(off, CHUNK)], idx_d)
                        pltpu.async_copy(table.at[idx_s], buf, sem).wait()
                        pltpu.sync_copy(buf, acc.at[idx_d], add=True)
                        return 0
                    lax.fori_loop(0, NCHUNKS, chunk, 0)
            plsc.subcore_barrier()

            for core in range(2):
                @pl.when(c == core)
                def _(out=outs[core * S + sl]):
                    pltpu.sync_copy(acc.at[pl.ds(row0, ROWS_PER_SUB)],
                                    out.at[pl.ds(row0, ROWS_PER_SUB)])
            if sl + 1 < S:
                plsc.subcore_barrier()

    return agg


_AGG2 = _make_agg(2)
_AGG4 = _make_agg(4)


def _make_deg():
    mesh = plsc.VectorSubcoreMesh(core_axis_name="c", subcore_axis_name="s")
    out_type = [jax.ShapeDtypeStruct((NPAD, DEGW), jnp.float32)
                for _ in range(2)]
    scratch_types = [
        pltpu.VMEM((CHUNK,), jnp.int32),
        pltpu.VMEM((CHUNK, DEGW), jnp.float32),
        pltpu.VMEM((ZROWS, DEGW), jnp.float32),
        pltpu.VMEM_SHARED((NPAD, DEGW), jnp.float32),
        pltpu.SemaphoreType.DMA,
    ]

    @functools.partial(pl.kernel, mesh=mesh, out_type=out_type,
                       scratch_types=scratch_types)
    def deg(src_hbm, dst_hbm, out_in, out_out, idx, ones, zbuf, acc, sem):
        c = lax.axis_index("c")
        s = lax.axis_index("s")
        row0 = s * ROWS_PER_SUB
        one16 = jnp.ones((16,), jnp.float32)
        zero16 = jnp.zeros((16,), jnp.float32)

        def fill(i, _):
            for j in range(DEGW // 16):
                ones[i, pl.ds(j * 16, 16)] = one16
            return 0
        lax.fori_loop(0, CHUNK, fill, 0)

        def zrow(i, _):
            for j in range(DEGW // 16):
                zbuf[i, pl.ds(j * 16, 16)] = zero16
            return 0
        lax.fori_loop(0, ZROWS, zrow, 0)

        for z in range(ROWS_PER_SUB // ZROWS):
            pltpu.sync_copy(zbuf, acc.at[pl.ds(row0 + z * ZROWS, ZROWS)])
        plsc.subcore_barrier()

        for core, idx_hbm in ((0, dst_hbm), (1, src_hbm)):
            @pl.when(c == core)
            def _(idx_hbm=idx_hbm):
                def chunk(i, _):
                    off = pl.multiple_of(s * EDGES_PER_SUB + i * CHUNK, CHUNK)
                    pltpu.sync_copy(idx_hbm.at[pl.ds(off, CHUNK)], idx)
                    pltpu.sync_copy(ones, acc.at[idx], add=True)
                    return 0
                lax.fori_loop(0, NCHUNKS, chunk, 0)
        plsc.subcore_barrier()

        for core, out in ((0, out_in), (1, out_out)):
            @pl.when(c == core)
            def _(out=out):
                pltpu.sync_copy(acc.at[pl.ds(row0, ROWS_PER_SUB)],
                                out.at[pl.ds(row0, ROWS_PER_SUB)])

    return deg


_DEG = _make_deg()


def _row_specs(shapes):
    return [pl.BlockSpec(sh, lambda i: (i, 0)) for sh in shapes]


def _full_spec(sh):
    return pl.BlockSpec(sh, lambda i: (0, 0))


def _slice_out(nsl):
    return ([pl.BlockSpec((R, LANE), lambda i: (i, 0))] * nsl,
            [jax.ShapeDtypeStruct((N, LANE), jnp.float32)] * nsl)


def _scale0(x, deg_out):
    def body(x_ref, d_ref, o0, o1):
        ns = lax.rsqrt(jnp.maximum(d_ref[...], 1.0))
        m = x_ref[...] * ns
        o0[...] = m[:, :LANE]
        o1[...] = m[:, LANE:]

    out_specs, out_shape = _slice_out(2)
    return pl.pallas_call(
        body, grid=(NBLK,),
        in_specs=_row_specs([(R, 256), (R, 1)]),
        out_specs=out_specs, out_shape=out_shape,
    )(x, deg_out)


def _gconv(aggs, deg_in, W, b):
    nin = len(aggs)

    def body(*refs):
        a_refs = refs[:nin]
        d_ref, w_ref, b_ref = refs[nin:nin + 3]
        outs = refs[nin + 3:]
        nd = lax.rsqrt(jnp.maximum(d_ref[...], 1.0))
        agg = jnp.concatenate([a[...] for a in a_refs], axis=1) * nd
        h = jnp.dot(agg, w_ref[...], preferred_element_type=jnp.float32)
        h = jnp.maximum(h + b_ref[...], 0.0)
        for j, o in enumerate(outs):
            o[...] = h[:, j * LANE:(j + 1) * LANE]

    out_specs, out_shape = _slice_out(4)
    return pl.pallas_call(
        body, grid=(NBLK,),
        in_specs=(_row_specs([(R, LANE)] * nin + [(R, 1)])
                  + [_full_spec(W.shape), _full_spec((1, 512))]),
        out_specs=out_specs, out_shape=out_shape,
    )(*aggs, deg_in, W, b.reshape(1, 512))


def _sage_mean_scaled(hs, aggs, deg_in, deg_out, Ws, Wn, b):
    def body(*refs):
        h_refs = refs[:4]
        a_refs = refs[4:8]
        di, do, ws_ref, wn_ref, b_ref = refs[8:13]
        outs = refs[13:]
        h = jnp.concatenate([r[...] for r in h_refs], axis=1)
        agg = jnp.concatenate([r[...] for r in a_refs], axis=1)
        hn = agg * (1.0 / jnp.maximum(di[...], 1.0))
        o = (jnp.dot(h, ws_ref[...], preferred_element_type=jnp.float32)
             + jnp.dot(hn, wn_ref[...], preferred_element_type=jnp.float32)
             + b_ref[...])
        o = jnp.maximum(o, 0.0) * lax.rsqrt(jnp.maximum(do[...], 1.0))
        for j, out in enumerate(outs):
            out[...] = o[:, j * LANE:(j + 1) * LANE]

    out_specs, out_shape = _slice_out(4)
    return pl.pallas_call(
        body, grid=(NBLK,),
        in_specs=(_row_specs([(R, LANE)] * 8 + [(R, 1), (R, 1)])
                  + [_full_spec((512, 512)), _full_spec((512, 512)),
                     _full_spec((1, 512))]),
        out_specs=out_specs, out_shape=out_shape,
    )(*hs, *aggs, deg_in, deg_out, Ws, Wn, b.reshape(1, 512))


def _sage_gcn_scaled(hs, aggs, deg_in, deg_out, Wn, b):
    def body(*refs):
        h_refs = refs[:4]
        a_refs = refs[4:8]
        di, do, wn_ref, b_ref = refs[8:12]
        outs = refs[12:]
        h = jnp.concatenate([r[...] for r in h_refs], axis=1)
        agg = jnp.concatenate([r[...] for r in a_refs], axis=1)
        hn = (agg + h) * (1.0 / (di[...] + 1.0))
        o = jnp.dot(hn, wn_ref[...], preferred_element_type=jnp.float32)
        o = jnp.maximum(o + b_ref[...], 0.0)
        o = o * lax.rsqrt(jnp.maximum(do[...], 1.0))
        for j, out in enumerate(outs):
            out[...] = o[:, j * LANE:(j + 1) * LANE]

    out_specs, out_shape = _slice_out(4)
    return pl.pallas_call(
        body, grid=(NBLK,),
        in_specs=(_row_specs([(R, LANE)] * 8 + [(R, 1), (R, 1)])
                  + [_full_spec((512, 512)), _full_spec((1, 512))]),
        out_specs=out_specs, out_shape=out_shape,
    )(*hs, *aggs, deg_in, deg_out, Wn, b.reshape(1, 512))


def _final(hs, aggs, deg_in, Ws, Wn, b, Wc, bc):
    def body(*refs):
        h_refs = refs[:4]
        a_refs = refs[4:8]
        di, ws_ref, wn_ref, b_ref, wc_ref, bc_ref = refs[8:14]
        out = refs[14]
        accr = refs[15]
        i = pl.program_id(0)
        h = jnp.concatenate([r[...] for r in h_refs], axis=1)
        agg = jnp.concatenate([r[...] for r in a_refs], axis=1)
        hn = agg * (1.0 / jnp.maximum(di[...], 1.0))
        o = (jnp.dot(h, ws_ref[...], preferred_element_type=jnp.float32)
             + jnp.dot(hn, wn_ref[...], preferred_element_type=jnp.float32)
             + b_ref[...])
        o = jnp.maximum(o, 0.0)
        part = jnp.sum(o, axis=0, keepdims=True)

        @pl.when(i == 0)
        def _():
            accr[...] = part

        @pl.when(i > 0)
        def _():
            accr[...] = accr[...] + part

        out[...] = (jnp.dot(accr[...] * (1.0 / N), wc_ref[...],
                            preferred_element_type=jnp.float32)
                    + bc_ref[...])

    return pl.pallas_call(
        body, grid=(NBLK,),
        in_specs=(_row_specs([(R, LANE)] * 8 + [(R, 1)])
                  + [_full_spec((512, 512)), _full_spec((512, 512)),
                     _full_spec((1, 512)), _full_spec((512, 40)),
                     _full_spec((1, 40))]),
        out_specs=pl.BlockSpec((1, 40), lambda i: (0, 0)),
        out_shape=jax.ShapeDtypeStruct((1, 40), jnp.float32),
        scratch_shapes=[pltpu.VMEM((1, 512), jnp.float32)],
    )(*hs, *aggs, deg_in, Ws, Wn, b.reshape(1, 512), Wc, bc.reshape(1, 40))


def kernel(x, edge_index, W1, b1, W2, b2, W3, b3, Ws1, Wn1, bS1, Wng, bg,
           Ws3, Wn3, bS3, Wc, bc):
    src = edge_index[0]
    dst = edge_index[1]

    deg_in_w, deg_out_w = _DEG(src, dst)
    deg_in = deg_in_w[:N, :1]
    deg_out = deg_out_w[:N, :1]

    m0 = _scale0(x, deg_out)
    a1 = _AGG2(m0[0], m0[1], src, dst)
    h1 = _gconv(a1, deg_in, W1, b1)

    a2 = _AGG4(*h1, src, dst)
    m2 = _sage_mean_scaled(h1, a2, deg_in, deg_out, Ws1, Wn1, bS1)

    a3 = _AGG4(*m2, src, dst)
    h3 = _gconv(a3, deg_in, W2, b2)

    a4 = _AGG4(*h3, src, dst)
    m4 = _sage_gcn_scaled(h3, a4, deg_in, deg_out, Wng, bg)

    a5 = _AGG4(*m4, src, dst)
    h5 = _gconv(a5, deg_in, W3, b3)

    a6 = _AGG4(*h5, src, dst)
    return _final(h5, a6, deg_in, Ws3, Wn3, bS3, Wc, bc)


# speedup vs baseline: 5.3078x; 1.9835x over previous
import functools

import jax
import jax.numpy as jnp
from jax import lax
from jax.experimental import pallas as pl
from jax.experimental.pallas import tpu as pltpu
from jax.experimental.pallas import tpu_sc as plsc

N = 10000
NPAD = 10240
E = 160000
LANE = 128
CHUNK = 40
NSUB = 16
EDGES_PER_SUB = E // NSUB
NCHUNKS = EDGES_PER_SUB // CHUNK
ROWS_PER_SUB = NPAD // NSUB
ZROWS = 64
DEGW = 128
R = 1000
NBLK = N // R


def _make_agg(nsl):
    S = nsl // 2
    G = 5
    NSTEP = NCHUNKS // G
    mesh = plsc.VectorSubcoreMesh(core_axis_name="c", subcore_axis_name="s")
    out_type = [jax.ShapeDtypeStruct((NPAD, LANE), jnp.float32)
                for _ in range(nsl)]
    scratch_types = (
        [pltpu.VMEM((CHUNK,), jnp.int32) for _ in range(G)]
        + [pltpu.VMEM((CHUNK,), jnp.int32) for _ in range(G)]
        + [pltpu.VMEM((CHUNK, LANE), jnp.float32) for _ in range(G)]
        + [pltpu.VMEM((ZROWS, LANE), jnp.float32)]
        + [pltpu.VMEM_SHARED((NPAD, LANE), jnp.float32)]
        + [pltpu.SemaphoreType.DMA for _ in range(3 * G)]
    )

    @functools.partial(pl.kernel, mesh=mesh, out_type=out_type,
                       scratch_types=scratch_types)
    def agg(*args):
        h_refs = args[:nsl]
        src_hbm = args[nsl]
        dst_hbm = args[nsl + 1]
        outs = args[nsl + 2:nsl + 2 + nsl]
        rest = args[nsl + 2 + nsl:]
        idx_s = rest[:G]
        idx_d = rest[G:2 * G]
        bufs = rest[2 * G:3 * G]
        zbuf = rest[3 * G]
        acc = rest[3 * G + 1]
        isem = rest[3 * G + 2:3 * G + 2 + G]
        gsem = rest[3 * G + 2 + G:3 * G + 2 + 2 * G]
        ssem = rest[3 * G + 2 + 2 * G:3 * G + 2 + 3 * G]
        c = lax.axis_index("c")
        s = lax.axis_index("s")
        row0 = s * ROWS_PER_SUB
        ebase = s * EDGES_PER_SUB

        zero16 = jnp.zeros((16,), jnp.float32)

        def zrow(i, _):
            for j in range(LANE // 16):
                zbuf[i, pl.ds(j * 16, 16)] = zero16
            return 0
        lax.fori_loop(0, ZROWS, zrow, 0)

        def start_idx(g, cc):
            off = pl.multiple_of(ebase + cc * CHUNK, CHUNK)
            pltpu.async_copy(src_hbm.at[pl.ds(off, CHUNK)], idx_s[g], isem[g])
            pltpu.async_copy(dst_hbm.at[pl.ds(off, CHUNK)], idx_d[g], isem[g])

        def wait_idx(g, cc):
            off = pl.multiple_of(ebase + cc * CHUNK, CHUNK)
            pltpu.make_async_copy(src_hbm.at[pl.ds(off, CHUNK)], idx_s[g],
                                  isem[g]).wait()
            pltpu.make_async_copy(dst_hbm.at[pl.ds(off, CHUNK)], idx_d[g],
                                  isem[g]).wait()

        def wait_scatter(g):
            pltpu.make_async_copy(bufs[g], acc.at[idx_d[g]], ssem[g]).wait()

        for sl in range(S):
            for z in range(ROWS_PER_SUB // ZROWS):
                pltpu.sync_copy(zbuf, acc.at[pl.ds(row0 + z * ZROWS, ZROWS)])
            plsc.subcore_barrier()

            for core in range(2):
                @pl.when(c == core)
                def _(table=h_refs[core * S + sl]):
                    for g in range(G):
                        start_idx(g, g)
                    gh = []
                    for g in range(G):
                        wait_idx(g, g)
                        gh.append(pltpu.async_copy(table.at[idx_s[g]],
                                                   bufs[g], gsem[g]))
                    for g in range(G):
                        gh[g].wait()
                        pltpu.async_copy(bufs[g], acc.at[idx_d[g]],
                                         ssem[g], add=True)

                    def step(t, _):
                        for g in range(G):
                            wait_scatter(g)
                            start_idx(g, t * G + g)
                        hh = []
                        for g in range(G):
                            wait_idx(g, t * G + g)
                            hh.append(pltpu.async_copy(table.at[idx_s[g]],
                                                       bufs[g], gsem[g]))
                        for g in range(G):
                            hh[g].wait()
                            pltpu.async_copy(bufs[g], acc.at[idx_d[g]],
                                             ssem[g], add=True)
                        return 0
                    lax.fori_loop(1, NSTEP, step, 0)
                    for g in range(G):
                        wait_scatter(g)
            plsc.subcore_barrier()

            for core in range(2):
                @pl.when(c == core)
                def _(out=outs[core * S + sl]):
                    pltpu.sync_copy(acc.at[pl.ds(row0, ROWS_PER_SUB)],
                                    out.at[pl.ds(row0, ROWS_PER_SUB)])
            if sl + 1 < S:
                plsc.subcore_barrier()

    return agg


_AGG2 = _make_agg(2)
_AGG4 = _make_agg(4)


def _make_deg():
    mesh = plsc.VectorSubcoreMesh(core_axis_name="c", subcore_axis_name="s")
    out_type = [jax.ShapeDtypeStruct((NPAD, DEGW), jnp.float32)
                for _ in range(2)]
    scratch_types = [
        pltpu.VMEM((CHUNK,), jnp.int32),
        pltpu.VMEM((CHUNK, DEGW), jnp.float32),
        pltpu.VMEM((ZROWS, DEGW), jnp.float32),
        pltpu.VMEM_SHARED((NPAD, DEGW), jnp.float32),
        pltpu.SemaphoreType.DMA,
    ]

    @functools.partial(pl.kernel, mesh=mesh, out_type=out_type,
                       scratch_types=scratch_types)
    def deg(src_hbm, dst_hbm, out_in, out_out, idx, ones, zbuf, acc, sem):
        c = lax.axis_index("c")
        s = lax.axis_index("s")
        row0 = s * ROWS_PER_SUB
        one16 = jnp.ones((16,), jnp.float32)
        zero16 = jnp.zeros((16,), jnp.float32)

        def fill(i, _):
            for j in range(DEGW // 16):
                ones[i, pl.ds(j * 16, 16)] = one16
            return 0
        lax.fori_loop(0, CHUNK, fill, 0)

        def zrow(i, _):
            for j in range(DEGW // 16):
                zbuf[i, pl.ds(j * 16, 16)] = zero16
            return 0
        lax.fori_loop(0, ZROWS, zrow, 0)

        for z in range(ROWS_PER_SUB // ZROWS):
            pltpu.sync_copy(zbuf, acc.at[pl.ds(row0 + z * ZROWS, ZROWS)])
        plsc.subcore_barrier()

        for core, idx_hbm in ((0, dst_hbm), (1, src_hbm)):
            @pl.when(c == core)
            def _(idx_hbm=idx_hbm):
                def chunk(i, _):
                    off = pl.multiple_of(s * EDGES_PER_SUB + i * CHUNK, CHUNK)
                    pltpu.sync_copy(idx_hbm.at[pl.ds(off, CHUNK)], idx)
                    pltpu.sync_copy(ones, acc.at[idx], add=True)
                    return 0
                lax.fori_loop(0, NCHUNKS, chunk, 0)
        plsc.subcore_barrier()

        for core, out in ((0, out_in), (1, out_out)):
            @pl.when(c == core)
            def _(out=out):
                pltpu.sync_copy(acc.at[pl.ds(row0, ROWS_PER_SUB)],
                                out.at[pl.ds(row0, ROWS_PER_SUB)])

    return deg


_DEG = _make_deg()


def _row_specs(shapes):
    return [pl.BlockSpec(sh, lambda i: (i, 0)) for sh in shapes]


def _full_spec(sh):
    return pl.BlockSpec(sh, lambda i: (0, 0))


def _slice_out(nsl):
    return ([pl.BlockSpec((R, LANE), lambda i: (i, 0))] * nsl,
            [jax.ShapeDtypeStruct((N, LANE), jnp.float32)] * nsl)


def _scale0(x, deg_out):
    def body(x_ref, d_ref, o0, o1):
        ns = lax.rsqrt(jnp.maximum(d_ref[...], 1.0))
        m = x_ref[...] * ns
        o0[...] = m[:, :LANE]
        o1[...] = m[:, LANE:]

    out_specs, out_shape = _slice_out(2)
    return pl.pallas_call(
        body, grid=(NBLK,),
        in_specs=_row_specs([(R, 256), (R, 1)]),
        out_specs=out_specs, out_shape=out_shape,
    )(x, deg_out)


def _gconv(aggs, deg_in, W, b):
    nin = len(aggs)

    def body(*refs):
        a_refs = refs[:nin]
        d_ref, w_ref, b_ref = refs[nin:nin + 3]
        outs = refs[nin + 3:]
        nd = lax.rsqrt(jnp.maximum(d_ref[...], 1.0))
        agg = jnp.concatenate([a[...] for a in a_refs], axis=1) * nd
        h = jnp.dot(agg, w_ref[...], preferred_element_type=jnp.float32)
        h = jnp.maximum(h + b_ref[...], 0.0)
        for j, o in enumerate(outs):
            o[...] = h[:, j * LANE:(j + 1) * LANE]

    out_specs, out_shape = _slice_out(4)
    return pl.pallas_call(
        body, grid=(NBLK,),
        in_specs=(_row_specs([(R, LANE)] * nin + [(R, 1)])
                  + [_full_spec(W.shape), _full_spec((1, 512))]),
        out_specs=out_specs, out_shape=out_shape,
    )(*aggs, deg_in, W, b.reshape(1, 512))


def _sage_mean_scaled(hs, aggs, deg_in, deg_out, Ws, Wn, b):
    def body(*refs):
        h_refs = refs[:4]
        a_refs = refs[4:8]
        di, do, ws_ref, wn_ref, b_ref = refs[8:13]
        outs = refs[13:]
        h = jnp.concatenate([r[...] for r in h_refs], axis=1)
        agg = jnp.concatenate([r[...] for r in a_refs], axis=1)
        hn = agg * (1.0 / jnp.maximum(di[...], 1.0))
        o = (jnp.dot(h, ws_ref[...], preferred_element_type=jnp.float32)
             + jnp.dot(hn, wn_ref[...], preferred_element_type=jnp.float32)
             + b_ref[...])
        o = jnp.maximum(o, 0.0) * lax.rsqrt(jnp.maximum(do[...], 1.0))
        for j, out in enumerate(outs):
            out[...] = o[:, j * LANE:(j + 1) * LANE]

    out_specs, out_shape = _slice_out(4)
    return pl.pallas_call(
        body, grid=(NBLK,),
        in_specs=(_row_specs([(R, LANE)] * 8 + [(R, 1), (R, 1)])
                  + [_full_spec((512, 512)), _full_spec((512, 512)),
                     _full_spec((1, 512))]),
        out_specs=out_specs, out_shape=out_shape,
    )(*hs, *aggs, deg_in, deg_out, Ws, Wn, b.reshape(1, 512))


def _sage_gcn_scaled(hs, aggs, deg_in, deg_out, Wn, b):
    def body(*refs):
        h_refs = refs[:4]
        a_refs = refs[4:8]
        di, do, wn_ref, b_ref = refs[8:12]
        outs = refs[12:]
        h = jnp.concatenate([r[...] for r in h_refs], axis=1)
        agg = jnp.concatenate([r[...] for r in a_refs], axis=1)
        hn = (agg + h) * (1.0 / (di[...] + 1.0))
        o = jnp.dot(hn, wn_ref[...], preferred_element_type=jnp.float32)
        o = jnp.maximum(o + b_ref[...], 0.0)
        o = o * lax.rsqrt(jnp.maximum(do[...], 1.0))
        for j, out in enumerate(outs):
            out[...] = o[:, j * LANE:(j + 1) * LANE]

    out_specs, out_shape = _slice_out(4)
    return pl.pallas_call(
        body, grid=(NBLK,),
        in_specs=(_row_specs([(R, LANE)] * 8 + [(R, 1), (R, 1)])
                  + [_full_spec((512, 512)), _full_spec((1, 512))]),
        out_specs=out_specs, out_shape=out_shape,
    )(*hs, *aggs, deg_in, deg_out, Wn, b.reshape(1, 512))


def _final(hs, aggs, deg_in, Ws, Wn, b, Wc, bc):
    def body(*refs):
        h_refs = refs[:4]
        a_refs = refs[4:8]
        di, ws_ref, wn_ref, b_ref, wc_ref, bc_ref = refs[8:14]
        out = refs[14]
        accr = refs[15]
        i = pl.program_id(0)
        h = jnp.concatenate([r[...] for r in h_refs], axis=1)
        agg = jnp.concatenate([r[...] for r in a_refs], axis=1)
        hn = agg * (1.0 / jnp.maximum(di[...], 1.0))
        o = (jnp.dot(h, ws_ref[...], preferred_element_type=jnp.float32)
             + jnp.dot(hn, wn_ref[...], preferred_element_type=jnp.float32)
             + b_ref[...])
        o = jnp.maximum(o, 0.0)
        part = jnp.sum(o, axis=0, keepdims=True)

        @pl.when(i == 0)
        def _():
            accr[...] = part

        @pl.when(i > 0)
        def _():
            accr[...] = accr[...] + part

        out[...] = (jnp.dot(accr[...] * (1.0 / N), wc_ref[...],
                            preferred_element_type=jnp.float32)
                    + bc_ref[...])

    return pl.pallas_call(
        body, grid=(NBLK,),
        in_specs=(_row_specs([(R, LANE)] * 8 + [(R, 1)])
                  + [_full_spec((512, 512)), _full_spec((512, 512)),
                     _full_spec((1, 512)), _full_spec((512, 40)),
                     _full_spec((1, 40))]),
        out_specs=pl.BlockSpec((1, 40), lambda i: (0, 0)),
        out_shape=jax.ShapeDtypeStruct((1, 40), jnp.float32),
        scratch_shapes=[pltpu.VMEM((1, 512), jnp.float32)],
    )(*hs, *aggs, deg_in, Ws, Wn, b.reshape(1, 512), Wc, bc.reshape(1, 40))


def kernel(x, edge_index, W1, b1, W2, b2, W3, b3, Ws1, Wn1, bS1, Wng, bg,
           Ws3, Wn3, bS3, Wc, bc):
    src = edge_index[0]
    dst = edge_index[1]

    deg_in_w, deg_out_w = _DEG(src, dst)
    deg_in = deg_in_w[:N, :1]
    deg_out = deg_out_w[:N, :1]

    m0 = _scale0(x, deg_out)
    a1 = _AGG2(m0[0], m0[1], src, dst)
    h1 = _gconv(a1, deg_in, W1, b1)

    a2 = _AGG4(*h1, src, dst)
    m2 = _sage_mean_scaled(h1, a2, deg_in, deg_out, Ws1, Wn1, bS1)

    a3 = _AGG4(*m2, src, dst)
    h3 = _gconv(a3, deg_in, W2, b2)

    a4 = _AGG4(*h3, src, dst)
    m4 = _sage_gcn_scaled(h3, a4, deg_in, deg_out, Wng, bg)

    a5 = _AGG4(*m4, src, dst)
    h5 = _gconv(a5, deg_in, W3, b3)

    a6 = _AGG4(*h5, src, dst)
    return _final(h5, a6, deg_in, Ws3, Wn3, bS3, Wc, bc)


# speedup vs baseline: 5.6597x; 1.0663x over previous
import functools

import jax
import jax.numpy as jnp
from jax import lax
from jax.experimental import pallas as pl
from jax.experimental.pallas import tpu as pltpu
from jax.experimental.pallas import tpu_sc as plsc

N = 10000
NPAD = 10240
E = 160000
LANE = 128
CHUNK = 40
NSUB = 16
EDGES_PER_SUB = E // NSUB
NCHUNKS = EDGES_PER_SUB // CHUNK
ROWS_PER_SUB = NPAD // NSUB
ZROWS = 64
DEGW = 128
R = 1000
NBLK = N // R


def _make_agg(nsl):
    S = nsl // 2
    G = 5
    NSTEP = NCHUNKS // G
    mesh = plsc.VectorSubcoreMesh(core_axis_name="c", subcore_axis_name="s")
    out_type = [jax.ShapeDtypeStruct((NPAD, LANE), jnp.float32)
                for _ in range(nsl)]
    scratch_types = (
        [pltpu.VMEM((CHUNK,), jnp.int32) for _ in range(G)]
        + [pltpu.VMEM((CHUNK,), jnp.int32) for _ in range(G)]
        + [pltpu.VMEM((CHUNK, LANE), jnp.float32) for _ in range(G)]
        + [pltpu.VMEM((ZROWS, LANE), jnp.float32)]
        + [pltpu.VMEM_SHARED((NPAD, LANE), jnp.float32)]
        + [pltpu.SemaphoreType.DMA for _ in range(3 * G)]
    )

    @functools.partial(pl.kernel, mesh=mesh, out_type=out_type,
                       scratch_types=scratch_types)
    def agg(*args):
        h_refs = args[:nsl]
        src_hbm = args[nsl]
        dst_hbm = args[nsl + 1]
        outs = args[nsl + 2:nsl + 2 + nsl]
        rest = args[nsl + 2 + nsl:]
        idx_s = rest[:G]
        idx_d = rest[G:2 * G]
        bufs = rest[2 * G:3 * G]
        zbuf = rest[3 * G]
        acc = rest[3 * G + 1]
        isem = rest[3 * G + 2:3 * G + 2 + G]
        gsem = rest[3 * G + 2 + G:3 * G + 2 + 2 * G]
        ssem = rest[3 * G + 2 + 2 * G:3 * G + 2 + 3 * G]
        c = lax.axis_index("c")
        s = lax.axis_index("s")
        row0 = s * ROWS_PER_SUB
        ebase = s * EDGES_PER_SUB

        zero16 = jnp.zeros((16,), jnp.float32)

        def zrow(i, _):
            for j in range(LANE // 16):
                zbuf[i, pl.ds(j * 16, 16)] = zero16
            return 0
        lax.fori_loop(0, ZROWS, zrow, 0)

        def start_idx(g, cc):
            off = pl.multiple_of(ebase + cc * CHUNK, CHUNK)
            pltpu.async_copy(src_hbm.at[pl.ds(off, CHUNK)], idx_s[g], isem[g])
            pltpu.async_copy(dst_hbm.at[pl.ds(off, CHUNK)], idx_d[g], isem[g])

        def wait_idx(g, cc):
            off = pl.multiple_of(ebase + cc * CHUNK, CHUNK)
            pltpu.make_async_copy(src_hbm.at[pl.ds(off, CHUNK)], idx_s[g],
                                  isem[g]).wait()
            pltpu.make_async_copy(dst_hbm.at[pl.ds(off, CHUNK)], idx_d[g],
                                  isem[g]).wait()

        def wait_scatter(g):
            pltpu.make_async_copy(bufs[g], acc.at[idx_d[g]], ssem[g]).wait()

        for sl in range(S):
            for z in range(ROWS_PER_SUB // ZROWS):
                pltpu.sync_copy(zbuf, acc.at[pl.ds(row0 + z * ZROWS, ZROWS)])
            plsc.subcore_barrier()

            for core in range(2):
                @pl.when(c == core)
                def _(table=h_refs[core * S + sl]):
                    for g in range(G):
                        start_idx(g, g)
                    gh = []
                    for g in range(G):
                        wait_idx(g, g)
                        gh.append(pltpu.async_copy(table.at[idx_s[g]],
                                                   bufs[g], gsem[g]))
                    for g in range(G):
                        gh[g].wait()
                        pltpu.async_copy(bufs[g], acc.at[idx_d[g]],
                                         ssem[g], add=True)

                    def step(t, _):
                        for g in range(G):
                            wait_scatter(g)
                            start_idx(g, t * G + g)
                        hh = []
                        for g in range(G):
                            wait_idx(g, t * G + g)
                            hh.append(pltpu.async_copy(table.at[idx_s[g]],
                                                       bufs[g], gsem[g]))
                        for g in range(G):
                            hh[g].wait()
                            pltpu.async_copy(bufs[g], acc.at[idx_d[g]],
                                             ssem[g], add=True)
                        return 0
                    lax.fori_loop(1, NSTEP, step, 0)
                    for g in range(G):
                        wait_scatter(g)
            plsc.subcore_barrier()

            for core in range(2):
                @pl.when(c == core)
                def _(out=outs[core * S + sl]):
                    pltpu.sync_copy(acc.at[pl.ds(row0, ROWS_PER_SUB)],
                                    out.at[pl.ds(row0, ROWS_PER_SUB)])
            if sl + 1 < S:
                plsc.subcore_barrier()

    return agg


_AGG2 = _make_agg(2)
_AGG4 = _make_agg(4)


DCHUNK = 80
NDCH = EDGES_PER_SUB // DCHUNK


def _make_deg():
    G = 5
    NSTEP = NDCH // G
    mesh = plsc.VectorSubcoreMesh(core_axis_name="c", subcore_axis_name="s")
    out_type = [jax.ShapeDtypeStruct((NPAD, LANE), jnp.float32)
                for _ in range(2)]
    scratch_types = (
        [pltpu.VMEM((DCHUNK,), jnp.int32) for _ in range(G)]
        + [pltpu.VMEM((DCHUNK, LANE), jnp.float32)]
        + [pltpu.VMEM((ZROWS, LANE), jnp.float32)]
        + [pltpu.VMEM_SHARED((NPAD, LANE), jnp.float32)]
        + [pltpu.SemaphoreType.DMA for _ in range(2 * G)]
    )

    @functools.partial(pl.kernel, mesh=mesh, out_type=out_type,
                       scratch_types=scratch_types)
    def deg(src_hbm, dst_hbm, out_in, out_out, *rest):
        idx = rest[:G]
        ones = rest[G]
        zbuf = rest[G + 1]
        acc = rest[G + 2]
        isem = rest[G + 3:G + 3 + G]
        ssem = rest[G + 3 + G:G + 3 + 2 * G]
        c = lax.axis_index("c")
        s = lax.axis_index("s")
        row0 = s * ROWS_PER_SUB
        ebase = s * EDGES_PER_SUB
        one16 = jnp.ones((16,), jnp.float32)
        zero16 = jnp.zeros((16,), jnp.float32)

        def orow(i, _):
            for j in range(LANE // 16):
                ones[i, pl.ds(j * 16, 16)] = one16
            return 0
        lax.fori_loop(0, DCHUNK, orow, 0)

        def zrow(i, _):
            for j in range(LANE // 16):
                zbuf[i, pl.ds(j * 16, 16)] = zero16
            return 0
        lax.fori_loop(0, ZROWS, zrow, 0)
        for z in range(ROWS_PER_SUB // ZROWS):
            pltpu.sync_copy(zbuf, acc.at[pl.ds(row0 + z * ZROWS, ZROWS)])
        plsc.subcore_barrier()

        for core, idx_hbm in ((0, dst_hbm), (1, src_hbm)):
            @pl.when(c == core)
            def _(idx_hbm=idx_hbm):
                def start_idx(g, cc):
                    off = pl.multiple_of(ebase + cc * DCHUNK, DCHUNK)
                    pltpu.async_copy(idx_hbm.at[pl.ds(off, DCHUNK)],
                                     idx[g], isem[g])

                def wait_idx(g, cc):
                    off = pl.multiple_of(ebase + cc * DCHUNK, DCHUNK)
                    pltpu.make_async_copy(idx_hbm.at[pl.ds(off, DCHUNK)],
                                          idx[g], isem[g]).wait()

                for g in range(G):
                    start_idx(g, g)
                for g in range(G):
                    wait_idx(g, g)
                    pltpu.async_copy(ones, acc.at[idx[g]], ssem[g], add=True)

                def step(t, _):
                    for g in range(G):
                        pltpu.make_async_copy(ones, acc.at[idx[g]],
                                              ssem[g]).wait()
                        start_idx(g, t * G + g)
                    for g in range(G):
                        wait_idx(g, t * G + g)
                        pltpu.async_copy(ones, acc.at[idx[g]],
                                         ssem[g], add=True)
                    return 0
                lax.fori_loop(1, NSTEP, step, 0)
                for g in range(G):
                    pltpu.make_async_copy(ones, acc.at[idx[g]],
                                          ssem[g]).wait()
        plsc.subcore_barrier()

        for core, out in ((0, out_in), (1, out_out)):
            @pl.when(c == core)
            def _(out=out):
                pltpu.sync_copy(acc.at[pl.ds(row0, ROWS_PER_SUB)],
                                out.at[pl.ds(row0, ROWS_PER_SUB)])

    return deg


_DEG = _make_deg()


def _row_specs(shapes):
    return [pl.BlockSpec(sh, lambda i: (i, 0)) for sh in shapes]


def _full_spec(sh):
    return pl.BlockSpec(sh, lambda i: (0, 0))


def _slice_out(nsl):
    return ([pl.BlockSpec((R, LANE), lambda i: (i, 0))] * nsl,
            [jax.ShapeDtypeStruct((N, LANE), jnp.float32)] * nsl)


def _scale0(x, deg_out):
    def body(x_ref, d_ref, o0, o1):
        ns = lax.rsqrt(jnp.maximum(d_ref[...], 1.0))
        m = x_ref[...] * ns
        o0[...] = m[:, :LANE]
        o1[...] = m[:, LANE:]

    out_specs, out_shape = _slice_out(2)
    return pl.pallas_call(
        body, grid=(NBLK,),
        in_specs=_row_specs([(R, 256), (R, 1)]),
        out_specs=out_specs, out_shape=out_shape,
    )(x, deg_out)


def _gconv(aggs, deg_in, W, b):
    nin = len(aggs)

    def body(*refs):
        a_refs = refs[:nin]
        d_ref, w_ref, b_ref = refs[nin:nin + 3]
        outs = refs[nin + 3:]
        nd = lax.rsqrt(jnp.maximum(d_ref[...], 1.0))
        agg = jnp.concatenate([a[...] for a in a_refs], axis=1) * nd
        h = jnp.dot(agg, w_ref[...], preferred_element_type=jnp.float32)
        h = jnp.maximum(h + b_ref[...], 0.0)
        for j, o in enumerate(outs):
            o[...] = h[:, j * LANE:(j + 1) * LANE]

    out_specs, out_shape = _slice_out(4)
    return pl.pallas_call(
        body, grid=(NBLK,),
        in_specs=(_row_specs([(R, LANE)] * nin + [(R, 1)])
                  + [_full_spec(W.shape), _full_spec((1, 512))]),
        out_specs=out_specs, out_shape=out_shape,
    )(*aggs, deg_in, W, b.reshape(1, 512))


def _sage_mean_scaled(hs, aggs, deg_in, deg_out, Ws, Wn, b):
    def body(*refs):
        h_refs = refs[:4]
        a_refs = refs[4:8]
        di, do, ws_ref, wn_ref, b_ref = refs[8:13]
        outs = refs[13:]
        h = jnp.concatenate([r[...] for r in h_refs], axis=1)
        agg = jnp.concatenate([r[...] for r in a_refs], axis=1)
        hn = agg * (1.0 / jnp.maximum(di[...], 1.0))
        o = (jnp.dot(h, ws_ref[...], preferred_element_type=jnp.float32)
             + jnp.dot(hn, wn_ref[...], preferred_element_type=jnp.float32)
             + b_ref[...])
        o = jnp.maximum(o, 0.0) * lax.rsqrt(jnp.maximum(do[...], 1.0))
        for j, out in enumerate(outs):
            out[...] = o[:, j * LANE:(j + 1) * LANE]

    out_specs, out_shape = _slice_out(4)
    return pl.pallas_call(
        body, grid=(NBLK,),
        in_specs=(_row_specs([(R, LANE)] * 8 + [(R, 1), (R, 1)])
                  + [_full_spec((512, 512)), _full_spec((512, 512)),
                     _full_spec((1, 512))]),
        out_specs=out_specs, out_shape=out_shape,
    )(*hs, *aggs, deg_in, deg_out, Ws, Wn, b.reshape(1, 512))


def _sage_gcn_scaled(hs, aggs, deg_in, deg_out, Wn, b):
    def body(*refs):
        h_refs = refs[:4]
        a_refs = refs[4:8]
        di, do, wn_ref, b_ref = refs[8:12]
        outs = refs[12:]
        h = jnp.concatenate([r[...] for r in h_refs], axis=1)
        agg = jnp.concatenate([r[...] for r in a_refs], axis=1)
        hn = (agg + h) * (1.0 / (di[...] + 1.0))
        o = jnp.dot(hn, wn_ref[...], preferred_element_type=jnp.float32)
        o = jnp.maximum(o + b_ref[...], 0.0)
        o = o * lax.rsqrt(jnp.maximum(do[...], 1.0))
        for j, out in enumerate(outs):
            out[...] = o[:, j * LANE:(j + 1) * LANE]

    out_specs, out_shape = _slice_out(4)
    return pl.pallas_call(
        body, grid=(NBLK,),
        in_specs=(_row_specs([(R, LANE)] * 8 + [(R, 1), (R, 1)])
                  + [_full_spec((512, 512)), _full_spec((1, 512))]),
        out_specs=out_specs, out_shape=out_shape,
    )(*hs, *aggs, deg_in, deg_out, Wn, b.reshape(1, 512))


def _final(hs, aggs, deg_in, Ws, Wn, b, Wc, bc):
    def body(*refs):
        h_refs = refs[:4]
        a_refs = refs[4:8]
        di, ws_ref, wn_ref, b_ref, wc_ref, bc_ref = refs[8:14]
        out = refs[14]
        accr = refs[15]
        i = pl.program_id(0)
        h = jnp.concatenate([r[...] for r in h_refs], axis=1)
        agg = jnp.concatenate([r[...] for r in a_refs], axis=1)
        hn = agg * (1.0 / jnp.maximum(di[...], 1.0))
        o = (jnp.dot(h, ws_ref[...], preferred_element_type=jnp.float32)
             + jnp.dot(hn, wn_ref[...], preferred_element_type=jnp.float32)
             + b_ref[...])
        o = jnp.maximum(o, 0.0)
        part = jnp.sum(o, axis=0, keepdims=True)

        @pl.when(i == 0)
        def _():
            accr[...] = part

        @pl.when(i > 0)
        def _():
            accr[...] = accr[...] + part

        out[...] = (jnp.dot(accr[...] * (1.0 / N), wc_ref[...],
                            preferred_element_type=jnp.float32)
                    + bc_ref[...])

    return pl.pallas_call(
        body, grid=(NBLK,),
        in_specs=(_row_specs([(R, LANE)] * 8 + [(R, 1)])
                  + [_full_spec((512, 512)), _full_spec((512, 512)),
                     _full_spec((1, 512)), _full_spec((512, 40)),
                     _full_spec((1, 40))]),
        out_specs=pl.BlockSpec((1, 40), lambda i: (0, 0)),
        out_shape=jax.ShapeDtypeStruct((1, 40), jnp.float32),
        scratch_shapes=[pltpu.VMEM((1, 512), jnp.float32)],
    )(*hs, *aggs, deg_in, Ws, Wn, b.reshape(1, 512), Wc, bc.reshape(1, 40))


def kernel(x, edge_index, W1, b1, W2, b2, W3, b3, Ws1, Wn1, bS1, Wng, bg,
           Ws3, Wn3, bS3, Wc, bc):
    src = edge_index[0]
    dst = edge_index[1]

    deg_in_w, deg_out_w = _DEG(src, dst)
    deg_in = deg_in_w[:N, :1]
    deg_out = deg_out_w[:N, :1]

    m0 = _scale0(x, deg_out)
    a1 = _AGG2(m0[0], m0[1], src, dst)
    h1 = _gconv(a1, deg_in, W1, b1)

    a2 = _AGG4(*h1, src, dst)
    m2 = _sage_mean_scaled(h1, a2, deg_in, deg_out, Ws1, Wn1, bS1)

    a3 = _AGG4(*m2, src, dst)
    h3 = _gconv(a3, deg_in, W2, b2)

    a4 = _AGG4(*h3, src, dst)
    m4 = _sage_gcn_scaled(h3, a4, deg_in, deg_out, Wng, bg)

    a5 = _AGG4(*m4, src, dst)
    h5 = _gconv(a5, deg_in, W3, b3)

    a6 = _AGG4(*h5, src, dst)
    return _final(h5, a6, deg_in, Ws3, Wn3, bS3, Wc, bc)


# speedup vs baseline: 6.6804x; 1.1803x over previous
import functools

import jax
import jax.numpy as jnp
from jax import lax
from jax.experimental import pallas as pl
from jax.experimental.pallas import tpu as pltpu
from jax.experimental.pallas import tpu_sc as plsc

N = 10000
NPAD = 10240
E = 160000
LANE = 128
CHUNK = 40
NSUB = 16
EDGES_PER_SUB = E // NSUB
NCHUNKS = EDGES_PER_SUB // CHUNK
ROWS_PER_SUB = NPAD // NSUB
ZROWS = 16
DEGW = 128
R = 1000
NBLK = N // R


def _make_agg(nsl):
    S = nsl // 2
    G = 5
    NSTEP = NCHUNKS // G
    mesh = plsc.VectorSubcoreMesh(core_axis_name="c", subcore_axis_name="s")
    out_type = [jax.ShapeDtypeStruct((NPAD, LANE), jnp.float32)
                for _ in range(nsl)]
    scratch_types = (
        [pltpu.VMEM((EDGES_PER_SUB,), jnp.int32) for _ in range(2)]
        + [pltpu.VMEM((CHUNK, LANE), jnp.float32) for _ in range(G)]
        + [pltpu.VMEM((ZROWS, LANE), jnp.float32)]
        + [pltpu.VMEM_SHARED((NPAD, LANE), jnp.float32)]
        + [pltpu.SemaphoreType.DMA for _ in range(2 + 2 * G)]
    )

    @functools.partial(pl.kernel, mesh=mesh, out_type=out_type,
                       scratch_types=scratch_types)
    def agg(*args):
        h_refs = args[:nsl]
        src_hbm = args[nsl]
        dst_hbm = args[nsl + 1]
        outs = args[nsl + 2:nsl + 2 + nsl]
        rest = args[nsl + 2 + nsl:]
        idx_s = rest[0]
        idx_d = rest[1]
        bufs = rest[2:2 + G]
        zbuf = rest[2 + G]
        acc = rest[3 + G]
        isem_s = rest[4 + G]
        isem_d = rest[5 + G]
        gsem = rest[6 + G:6 + 2 * G]
        ssem = rest[6 + 2 * G:6 + 3 * G]
        c = lax.axis_index("c")
        s = lax.axis_index("s")
        row0 = s * ROWS_PER_SUB
        ebase = pl.multiple_of(s * EDGES_PER_SUB, EDGES_PER_SUB)

        cp_s = pltpu.async_copy(src_hbm.at[pl.ds(ebase, EDGES_PER_SUB)],
                                idx_s, isem_s)
        cp_d = pltpu.async_copy(dst_hbm.at[pl.ds(ebase, EDGES_PER_SUB)],
                                idx_d, isem_d)

        zero16 = jnp.zeros((16,), jnp.float32)

        def zrow(i, _):
            for j in range(LANE // 16):
                zbuf[i, pl.ds(j * 16, 16)] = zero16
            return 0
        lax.fori_loop(0, ZROWS, zrow, 0)
        cp_s.wait()
        cp_d.wait()

        def sidx(cc):
            return idx_s.at[pl.ds(pl.multiple_of(cc * CHUNK, CHUNK), CHUNK)]

        def didx(cc):
            return idx_d.at[pl.ds(pl.multiple_of(cc * CHUNK, CHUNK), CHUNK)]

        for sl in range(S):
            for z in range(ROWS_PER_SUB // ZROWS):
                pltpu.sync_copy(zbuf, acc.at[pl.ds(row0 + z * ZROWS, ZROWS)])
            plsc.subcore_barrier()

            for core in range(2):
                @pl.when(c == core)
                def _(table=h_refs[core * S + sl]):
                    gh = []
                    for g in range(G):
                        gh.append(pltpu.async_copy(table.at[sidx(g)],
                                                   bufs[g], gsem[g]))
                    for g in range(G):
                        gh[g].wait()
                        pltpu.async_copy(bufs[g], acc.at[didx(g)],
                                         ssem[g], add=True)

                    def step(t, _):
                        for g in range(G):
                            pltpu.make_async_copy(
                                bufs[g], acc.at[didx((t - 1) * G + g)],
                                ssem[g]).wait()
                            pltpu.async_copy(table.at[sidx(t * G + g)],
                                             bufs[g], gsem[g])
                        for g in range(G):
                            pltpu.make_async_copy(
                                table.at[sidx(t * G + g)], bufs[g],
                                gsem[g]).wait()
                            pltpu.async_copy(bufs[g], acc.at[didx(t * G + g)],
                                             ssem[g], add=True)
                        return 0
                    lax.fori_loop(1, NSTEP, step, 0)
                    for g in range(G):
                        pltpu.make_async_copy(
                            bufs[g], acc.at[didx((NSTEP - 1) * G + g)],
                            ssem[g]).wait()
            plsc.subcore_barrier()

            for core in range(2):
                @pl.when(c == core)
                def _(out=outs[core * S + sl]):
                    pltpu.sync_copy(acc.at[pl.ds(row0, ROWS_PER_SUB)],
                                    out.at[pl.ds(row0, ROWS_PER_SUB)])
            if sl + 1 < S:
                plsc.subcore_barrier()

    return agg


_AGG2 = _make_agg(2)
_AGG4 = _make_agg(4)


DCHUNK = 80
NDCH = EDGES_PER_SUB // DCHUNK


def _make_deg():
    G = 5
    NSTEP = NDCH // G
    mesh = plsc.VectorSubcoreMesh(core_axis_name="c", subcore_axis_name="s")
    out_type = [jax.ShapeDtypeStruct((NPAD, LANE), jnp.float32)
                for _ in range(2)]
    scratch_types = (
        [pltpu.VMEM((DCHUNK,), jnp.int32) for _ in range(G)]
        + [pltpu.VMEM((DCHUNK, LANE), jnp.float32)]
        + [pltpu.VMEM((ZROWS, LANE), jnp.float32)]
        + [pltpu.VMEM_SHARED((NPAD, LANE), jnp.float32)]
        + [pltpu.SemaphoreType.DMA for _ in range(2 * G)]
    )

    @functools.partial(pl.kernel, mesh=mesh, out_type=out_type,
                       scratch_types=scratch_types)
    def deg(src_hbm, dst_hbm, out_in, out_out, *rest):
        idx = rest[:G]
        ones = rest[G]
        zbuf = rest[G + 1]
        acc = rest[G + 2]
        isem = rest[G + 3:G + 3 + G]
        ssem = rest[G + 3 + G:G + 3 + 2 * G]
        c = lax.axis_index("c")
        s = lax.axis_index("s")
        row0 = s * ROWS_PER_SUB
        ebase = s * EDGES_PER_SUB
        one16 = jnp.ones((16,), jnp.float32)
        zero16 = jnp.zeros((16,), jnp.float32)

        def orow(i, _):
            for j in range(LANE // 16):
                ones[i, pl.ds(j * 16, 16)] = one16
            return 0
        lax.fori_loop(0, DCHUNK, orow, 0)

        def zrow(i, _):
            for j in range(LANE // 16):
                zbuf[i, pl.ds(j * 16, 16)] = zero16
            return 0
        lax.fori_loop(0, ZROWS, zrow, 0)
        for z in range(ROWS_PER_SUB // ZROWS):
            pltpu.sync_copy(zbuf, acc.at[pl.ds(row0 + z * ZROWS, ZROWS)])
        plsc.subcore_barrier()

        for core, idx_hbm in ((0, dst_hbm), (1, src_hbm)):
            @pl.when(c == core)
            def _(idx_hbm=idx_hbm):
                def start_idx(g, cc):
                    off = pl.multiple_of(ebase + cc * DCHUNK, DCHUNK)
                    pltpu.async_copy(idx_hbm.at[pl.ds(off, DCHUNK)],
                                     idx[g], isem[g])

                def wait_idx(g, cc):
                    off = pl.multiple_of(ebase + cc * DCHUNK, DCHUNK)
                    pltpu.make_async_copy(idx_hbm.at[pl.ds(off, DCHUNK)],
                                          idx[g], isem[g]).wait()

                for g in range(G):
                    start_idx(g, g)
                for g in range(G):
                    wait_idx(g, g)
                    pltpu.async_copy(ones, acc.at[idx[g]], ssem[g], add=True)

                def step(t, _):
                    for g in range(G):
                        pltpu.make_async_copy(ones, acc.at[idx[g]],
                                              ssem[g]).wait()
                        start_idx(g, t * G + g)
                    for g in range(G):
                        wait_idx(g, t * G + g)
                        pltpu.async_copy(ones, acc.at[idx[g]],
                                         ssem[g], add=True)
                    return 0
                lax.fori_loop(1, NSTEP, step, 0)
                for g in range(G):
                    pltpu.make_async_copy(ones, acc.at[idx[g]],
                                          ssem[g]).wait()
        plsc.subcore_barrier()

        for core, out in ((0, out_in), (1, out_out)):
            @pl.when(c == core)
            def _(out=out):
                pltpu.sync_copy(acc.at[pl.ds(row0, ROWS_PER_SUB)],
                                out.at[pl.ds(row0, ROWS_PER_SUB)])

    return deg


_DEG = _make_deg()


def _row_specs(shapes):
    return [pl.BlockSpec(sh, lambda i: (i, 0)) for sh in shapes]


def _full_spec(sh):
    return pl.BlockSpec(sh, lambda i: (0, 0))


def _slice_out(nsl):
    return ([pl.BlockSpec((R, LANE), lambda i: (i, 0))] * nsl,
            [jax.ShapeDtypeStruct((N, LANE), jnp.float32)] * nsl)


def _scale0(x, deg_out):
    def body(x_ref, d_ref, o0, o1):
        ns = lax.rsqrt(jnp.maximum(d_ref[...], 1.0))
        m = x_ref[...] * ns
        o0[...] = m[:, :LANE]
        o1[...] = m[:, LANE:]

    out_specs, out_shape = _slice_out(2)
    return pl.pallas_call(
        body, grid=(NBLK,),
        in_specs=_row_specs([(R, 256), (R, 1)]),
        out_specs=out_specs, out_shape=out_shape,
    )(x, deg_out)


def _gconv(aggs, deg_in, W, b):
    nin = len(aggs)

    def body(*refs):
        a_refs = refs[:nin]
        d_ref, w_ref, b_ref = refs[nin:nin + 3]
        outs = refs[nin + 3:]
        nd = lax.rsqrt(jnp.maximum(d_ref[...], 1.0))
        agg = jnp.concatenate([a[...] for a in a_refs], axis=1) * nd
        h = jnp.dot(agg, w_ref[...], preferred_element_type=jnp.float32)
        h = jnp.maximum(h + b_ref[...], 0.0)
        for j, o in enumerate(outs):
            o[...] = h[:, j * LANE:(j + 1) * LANE]

    out_specs, out_shape = _slice_out(4)
    return pl.pallas_call(
        body, grid=(NBLK,),
        in_specs=(_row_specs([(R, LANE)] * nin + [(R, 1)])
                  + [_full_spec(W.shape), _full_spec((1, 512))]),
        out_specs=out_specs, out_shape=out_shape,
    )(*aggs, deg_in, W, b.reshape(1, 512))


def _sage_mean_scaled(hs, aggs, deg_in, deg_out, Ws, Wn, b):
    def body(*refs):
        h_refs = refs[:4]
        a_refs = refs[4:8]
        di, do, ws_ref, wn_ref, b_ref = refs[8:13]
        outs = refs[13:]
        h = jnp.concatenate([r[...] for r in h_refs], axis=1)
        agg = jnp.concatenate([r[...] for r in a_refs], axis=1)
        hn = agg * (1.0 / jnp.maximum(di[...], 1.0))
        o = (jnp.dot(h, ws_ref[...], preferred_element_type=jnp.float32)
             + jnp.dot(hn, wn_ref[...], preferred_element_type=jnp.float32)
             + b_ref[...])
        o = jnp.maximum(o, 0.0) * lax.rsqrt(jnp.maximum(do[...], 1.0))
        for j, out in enumerate(outs):
            out[...] = o[:, j * LANE:(j + 1) * LANE]

    out_specs, out_shape = _slice_out(4)
    return pl.pallas_call(
        body, grid=(NBLK,),
        in_specs=(_row_specs([(R, LANE)] * 8 + [(R, 1), (R, 1)])
                  + [_full_spec((512, 512)), _full_spec((512, 512)),
                     _full_spec((1, 512))]),
        out_specs=out_specs, out_shape=out_shape,
    )(*hs, *aggs, deg_in, deg_out, Ws, Wn, b.reshape(1, 512))


def _sage_gcn_scaled(hs, aggs, deg_in, deg_out, Wn, b):
    def body(*refs):
        h_refs = refs[:4]
        a_refs = refs[4:8]
        di, do, wn_ref, b_ref = refs[8:12]
        outs = refs[12:]
        h = jnp.concatenate([r[...] for r in h_refs], axis=1)
        agg = jnp.concatenate([r[...] for r in a_refs], axis=1)
        hn = (agg + h) * (1.0 / (di[...] + 1.0))
        o = jnp.dot(hn, wn_ref[...], preferred_element_type=jnp.float32)
        o = jnp.maximum(o + b_ref[...], 0.0)
        o = o * lax.rsqrt(jnp.maximum(do[...], 1.0))
        for j, out in enumerate(outs):
            out[...] = o[:, j * LANE:(j + 1) * LANE]

    out_specs, out_shape = _slice_out(4)
    return pl.pallas_call(
        body, grid=(NBLK,),
        in_specs=(_row_specs([(R, LANE)] * 8 + [(R, 1), (R, 1)])
                  + [_full_spec((512, 512)), _full_spec((1, 512))]),
        out_specs=out_specs, out_shape=out_shape,
    )(*hs, *aggs, deg_in, deg_out, Wn, b.reshape(1, 512))


def _final(hs, aggs, deg_in, Ws, Wn, b, Wc, bc):
    def body(*refs):
        h_refs = refs[:4]
        a_refs = refs[4:8]
        di, ws_ref, wn_ref, b_ref, wc_ref, bc_ref = refs[8:14]
        out = refs[14]
        accr = refs[15]
        i = pl.program_id(0)
        h = jnp.concatenate([r[...] for r in h_refs], axis=1)
        agg = jnp.concatenate([r[...] for r in a_refs], axis=1)
        hn = agg * (1.0 / jnp.maximum(di[...], 1.0))
        o = (jnp.dot(h, ws_ref[...], preferred_element_type=jnp.float32)
             + jnp.dot(hn, wn_ref[...], preferred_element_type=jnp.float32)
             + b_ref[...])
        o = jnp.maximum(o, 0.0)
        part = jnp.sum(o, axis=0, keepdims=True)

        @pl.when(i == 0)
        def _():
            accr[...] = part

        @pl.when(i > 0)
        def _():
            accr[...] = accr[...] + part

        out[...] = (jnp.dot(accr[...] * (1.0 / N), wc_ref[...],
                            preferred_element_type=jnp.float32)
                    + bc_ref[...])

    return pl.pallas_call(
        body, grid=(NBLK,),
        in_specs=(_row_specs([(R, LANE)] * 8 + [(R, 1)])
                  + [_full_spec((512, 512)), _full_spec((512, 512)),
                     _full_spec((1, 512)), _full_spec((512, 40)),
                     _full_spec((1, 40))]),
        out_specs=pl.BlockSpec((1, 40), lambda i: (0, 0)),
        out_shape=jax.ShapeDtypeStruct((1, 40), jnp.float32),
        scratch_shapes=[pltpu.VMEM((1, 512), jnp.float32)],
    )(*hs, *aggs, deg_in, Ws, Wn, b.reshape(1, 512), Wc, bc.reshape(1, 40))


def kernel(x, edge_index, W1, b1, W2, b2, W3, b3, Ws1, Wn1, bS1, Wng, bg,
           Ws3, Wn3, bS3, Wc, bc):
    src = edge_index[0]
    dst = edge_index[1]

    deg_in_w, deg_out_w = _DEG(src, dst)
    deg_in = deg_in_w[:N, :1]
    deg_out = deg_out_w[:N, :1]

    m0 = _scale0(x, deg_out)
    a1 = _AGG2(m0[0], m0[1], src, dst)
    h1 = _gconv(a1, deg_in, W1, b1)

    a2 = _AGG4(*h1, src, dst)
    m2 = _sage_mean_scaled(h1, a2, deg_in, deg_out, Ws1, Wn1, bS1)

    a3 = _AGG4(*m2, src, dst)
    h3 = _gconv(a3, deg_in, W2, b2)

    a4 = _AGG4(*h3, src, dst)
    m4 = _sage_gcn_scaled(h3, a4, deg_in, deg_out, Wng, bg)

    a5 = _AGG4(*m4, src, dst)
    h5 = _gconv(a5, deg_in, W3, b3)

    a6 = _AGG4(*h5, src, dst)
    return _final(h5, a6, deg_in, Ws3, Wn3, bS3, Wc, bc)


# speedup vs baseline: 6.6831x; 1.0004x over previous
import functools

import jax
import jax.numpy as jnp
from jax import lax
from jax.experimental import pallas as pl
from jax.experimental.pallas import tpu as pltpu
from jax.experimental.pallas import tpu_sc as plsc

N = 10000
NPAD = 10240
E = 160000
LANE = 128
CHUNK = 40
NSUB = 16
EDGES_PER_SUB = E // NSUB
NCHUNKS = EDGES_PER_SUB // CHUNK
ROWS_PER_SUB = NPAD // NSUB
ZROWS = 16
DEGW = 128
R = 1000
NBLK = N // R


def _make_agg(nsl):
    S = nsl // 2
    G = 5
    NSTEP = NCHUNKS // G
    mesh = plsc.VectorSubcoreMesh(core_axis_name="c", subcore_axis_name="s")
    out_type = [jax.ShapeDtypeStruct((NPAD, LANE), jnp.float32)
                for _ in range(nsl)]
    scratch_types = (
        [pltpu.VMEM((EDGES_PER_SUB,), jnp.int32) for _ in range(2)]
        + [pltpu.VMEM((CHUNK, LANE), jnp.float32) for _ in range(G)]
        + [pltpu.VMEM((ZROWS, LANE), jnp.float32)]
        + [pltpu.VMEM_SHARED((NPAD, LANE), jnp.float32)]
        + [pltpu.SemaphoreType.DMA for _ in range(2 + 2 * G)]
    )

    @functools.partial(pl.kernel, mesh=mesh, out_type=out_type,
                       scratch_types=scratch_types)
    def agg(*args):
        h_refs = args[:nsl]
        src_hbm = args[nsl]
        dst_hbm = args[nsl + 1]
        outs = args[nsl + 2:nsl + 2 + nsl]
        rest = args[nsl + 2 + nsl:]
        idx_s = rest[0]
        idx_d = rest[1]
        bufs = rest[2:2 + G]
        zbuf = rest[2 + G]
        acc = rest[3 + G]
        isem_s = rest[4 + G]
        isem_d = rest[5 + G]
        gsem = rest[6 + G:6 + 2 * G]
        ssem = rest[6 + 2 * G:6 + 3 * G]
        c = lax.axis_index("c")
        s = lax.axis_index("s")
        row0 = s * ROWS_PER_SUB
        ebase = pl.multiple_of(s * EDGES_PER_SUB, EDGES_PER_SUB)

        cp_s = pltpu.async_copy(src_hbm.at[pl.ds(ebase, EDGES_PER_SUB)],
                                idx_s, isem_s)
        cp_d = pltpu.async_copy(dst_hbm.at[pl.ds(ebase, EDGES_PER_SUB)],
                                idx_d, isem_d)

        zero16 = jnp.zeros((16,), jnp.float32)

        def zrow(i, _):
            for j in range(LANE // 16):
                zbuf[i, pl.ds(j * 16, 16)] = zero16
            return 0
        lax.fori_loop(0, ZROWS, zrow, 0)
        cp_s.wait()
        cp_d.wait()

        def sidx(cc):
            return idx_s.at[pl.ds(pl.multiple_of(cc * CHUNK, CHUNK), CHUNK)]

        def didx(cc):
            return idx_d.at[pl.ds(pl.multiple_of(cc * CHUNK, CHUNK), CHUNK)]

        for sl in range(S):
            for z in range(ROWS_PER_SUB // ZROWS):
                pltpu.sync_copy(zbuf, acc.at[pl.ds(row0 + z * ZROWS, ZROWS)])
            plsc.subcore_barrier()

            for core in range(2):
                @pl.when(c == core)
                def _(table=h_refs[core * S + sl]):
                    gh = []
                    for g in range(G):
                        gh.append(pltpu.async_copy(table.at[sidx(g)],
                                                   bufs[g], gsem[g]))
                    for g in range(G):
                        gh[g].wait()
                        pltpu.async_copy(bufs[g], acc.at[didx(g)],
                                         ssem[g], add=True)

                    def step(t, _):
                        for g in range(G):
                            pltpu.make_async_copy(
                                bufs[g], acc.at[didx((t - 1) * G + g)],
                                ssem[g]).wait()
                            pltpu.async_copy(table.at[sidx(t * G + g)],
                                             bufs[g], gsem[g])
                        for g in range(G):
                            pltpu.make_async_copy(
                                table.at[sidx(t * G + g)], bufs[g],
                                gsem[g]).wait()
                            pltpu.async_copy(bufs[g], acc.at[didx(t * G + g)],
                                             ssem[g], add=True)
                        return 0
                    lax.fori_loop(1, NSTEP, step, 0)
                    for g in range(G):
                        pltpu.make_async_copy(
                            bufs[g], acc.at[didx((NSTEP - 1) * G + g)],
                            ssem[g]).wait()
            plsc.subcore_barrier()

            for core in range(2):
                @pl.when(c == core)
                def _(out=outs[core * S + sl]):
                    pltpu.sync_copy(acc.at[pl.ds(row0, ROWS_PER_SUB)],
                                    out.at[pl.ds(row0, ROWS_PER_SUB)])
            if sl + 1 < S:
                plsc.subcore_barrier()

    return agg


_AGG2 = _make_agg(2)
_AGG4 = _make_agg(4)


DCHUNK = 80
NDCH = EDGES_PER_SUB // DCHUNK


def _make_deg():
    G = 5
    NSTEP = NDCH // G
    mesh = plsc.VectorSubcoreMesh(core_axis_name="c", subcore_axis_name="s")
    out_type = [jax.ShapeDtypeStruct((NPAD, LANE), jnp.float32)
                for _ in range(2)]
    scratch_types = (
        [pltpu.VMEM((EDGES_PER_SUB,), jnp.int32)]
        + [pltpu.VMEM((DCHUNK, LANE), jnp.float32)]
        + [pltpu.VMEM((ZROWS, LANE), jnp.float32)]
        + [pltpu.VMEM_SHARED((NPAD, LANE), jnp.float32)]
        + [pltpu.SemaphoreType.DMA for _ in range(1 + G)]
    )

    @functools.partial(pl.kernel, mesh=mesh, out_type=out_type,
                       scratch_types=scratch_types)
    def deg(src_hbm, dst_hbm, out_in, out_out, *rest):
        idx = rest[0]
        ones = rest[1]
        zbuf = rest[2]
        acc = rest[3]
        isem = rest[4]
        ssem = rest[5:5 + G]
        c = lax.axis_index("c")
        s = lax.axis_index("s")
        row0 = s * ROWS_PER_SUB
        ebase = pl.multiple_of(s * EDGES_PER_SUB, EDGES_PER_SUB)
        one16 = jnp.ones((16,), jnp.float32)
        zero16 = jnp.zeros((16,), jnp.float32)

        for core, idx_hbm in ((0, dst_hbm), (1, src_hbm)):
            @pl.when(c == core)
            def _(idx_hbm=idx_hbm):
                pltpu.async_copy(idx_hbm.at[pl.ds(ebase, EDGES_PER_SUB)],
                                 idx, isem)

        def orow(i, _):
            for j in range(LANE // 16):
                ones[i, pl.ds(j * 16, 16)] = one16
            return 0
        lax.fori_loop(0, DCHUNK, orow, 0)

        def zrow(i, _):
            for j in range(LANE // 16):
                zbuf[i, pl.ds(j * 16, 16)] = zero16
            return 0
        lax.fori_loop(0, ZROWS, zrow, 0)
        for z in range(ROWS_PER_SUB // ZROWS):
            pltpu.sync_copy(zbuf, acc.at[pl.ds(row0 + z * ZROWS, ZROWS)])
        for core, idx_hbm in ((0, dst_hbm), (1, src_hbm)):
            @pl.when(c == core)
            def _(idx_hbm=idx_hbm):
                pltpu.make_async_copy(idx_hbm.at[pl.ds(ebase, EDGES_PER_SUB)],
                                      idx, isem).wait()
        plsc.subcore_barrier()

        def cidx(cc):
            return idx.at[pl.ds(pl.multiple_of(cc * DCHUNK, DCHUNK), DCHUNK)]

        for g in range(G):
            pltpu.async_copy(ones, acc.at[cidx(g)], ssem[g], add=True)

        def step(t, _):
            for g in range(G):
                pltpu.make_async_copy(ones, acc.at[cidx((t - 1) * G + g)],
                                      ssem[g]).wait()
                pltpu.async_copy(ones, acc.at[cidx(t * G + g)],
                                 ssem[g], add=True)
            return 0
        lax.fori_loop(1, NSTEP, step, 0)
        for g in range(G):
            pltpu.make_async_copy(ones, acc.at[cidx((NSTEP - 1) * G + g)],
                                  ssem[g]).wait()
        plsc.subcore_barrier()

        for core, out in ((0, out_in), (1, out_out)):
            @pl.when(c == core)
            def _(out=out):
                pltpu.sync_copy(acc.at[pl.ds(row0, ROWS_PER_SUB)],
                                out.at[pl.ds(row0, ROWS_PER_SUB)])

    return deg


_DEG = _make_deg()


def _row_specs(shapes):
    return [pl.BlockSpec(sh, lambda i: (i, 0)) for sh in shapes]


def _full_spec(sh):
    return pl.BlockSpec(sh, lambda i: (0, 0))


def _slice_out(nsl):
    return ([pl.BlockSpec((R, LANE), lambda i: (i, 0))] * nsl,
            [jax.ShapeDtypeStruct((N, LANE), jnp.float32)] * nsl)


def _scale0(x, deg_out):
    def body(x_ref, d_ref, o0, o1):
        ns = lax.rsqrt(jnp.maximum(d_ref[...], 1.0))
        m = x_ref[...] * ns
        o0[...] = m[:, :LANE]
        o1[...] = m[:, LANE:]

    out_specs, out_shape = _slice_out(2)
    return pl.pallas_call(
        body, grid=(NBLK,),
        in_specs=_row_specs([(R, 256), (R, 1)]),
        out_specs=out_specs, out_shape=out_shape,
    )(x, deg_out)


def _gconv(aggs, deg_in, W, b):
    nin = len(aggs)

    def body(*refs):
        a_refs = refs[:nin]
        d_ref, w_ref, b_ref = refs[nin:nin + 3]
        outs = refs[nin + 3:]
        nd = lax.rsqrt(jnp.maximum(d_ref[...], 1.0))
        agg = jnp.concatenate([a[...] for a in a_refs], axis=1) * nd
        h = jnp.dot(agg, w_ref[...], preferred_element_type=jnp.float32)
        h = jnp.maximum(h + b_ref[...], 0.0)
        for j, o in enumerate(outs):
            o[...] = h[:, j * LANE:(j + 1) * LANE]

    out_specs, out_shape = _slice_out(4)
    return pl.pallas_call(
        body, grid=(NBLK,),
        in_specs=(_row_specs([(R, LANE)] * nin + [(R, 1)])
                  + [_full_spec(W.shape), _full_spec((1, 512))]),
        out_specs=out_specs, out_shape=out_shape,
    )(*aggs, deg_in, W, b.reshape(1, 512))


def _sage_mean_scaled(hs, aggs, deg_in, deg_out, Ws, Wn, b):
    def body(*refs):
        h_refs = refs[:4]
        a_refs = refs[4:8]
        di, do, ws_ref, wn_ref, b_ref = refs[8:13]
        outs = refs[13:]
        h = jnp.concatenate([r[...] for r in h_refs], axis=1)
        agg = jnp.concatenate([r[...] for r in a_refs], axis=1)
        hn = agg * (1.0 / jnp.maximum(di[...], 1.0))
        o = (jnp.dot(h, ws_ref[...], preferred_element_type=jnp.float32)
             + jnp.dot(hn, wn_ref[...], preferred_element_type=jnp.float32)
             + b_ref[...])
        o = jnp.maximum(o, 0.0) * lax.rsqrt(jnp.maximum(do[...], 1.0))
        for j, out in enumerate(outs):
            out[...] = o[:, j * LANE:(j + 1) * LANE]

    out_specs, out_shape = _slice_out(4)
    return pl.pallas_call(
        body, grid=(NBLK,),
        in_specs=(_row_specs([(R, LANE)] * 8 + [(R, 1), (R, 1)])
                  + [_full_spec((512, 512)), _full_spec((512, 512)),
                     _full_spec((1, 512))]),
        out_specs=out_specs, out_shape=out_shape,
    )(*hs, *aggs, deg_in, deg_out, Ws, Wn, b.reshape(1, 512))


def _sage_gcn_scaled(hs, aggs, deg_in, deg_out, Wn, b):
    def body(*refs):
        h_refs = refs[:4]
        a_refs = refs[4:8]
        di, do, wn_ref, b_ref = refs[8:12]
        outs = refs[12:]
        h = jnp.concatenate([r[...] for r in h_refs], axis=1)
        agg = jnp.concatenate([r[...] for r in a_refs], axis=1)
        hn = (agg + h) * (1.0 / (di[...] + 1.0))
        o = jnp.dot(hn, wn_ref[...], preferred_element_type=jnp.float32)
        o = jnp.maximum(o + b_ref[...], 0.0)
        o = o * lax.rsqrt(jnp.maximum(do[...], 1.0))
        for j, out in enumerate(outs):
            out[...] = o[:, j * LANE:(j + 1) * LANE]

    out_specs, out_shape = _slice_out(4)
    return pl.pallas_call(
        body, grid=(NBLK,),
        in_specs=(_row_specs([(R, LANE)] * 8 + [(R, 1), (R, 1)])
                  + [_full_spec((512, 512)), _full_spec((1, 512))]),
        out_specs=out_specs, out_shape=out_shape,
    )(*hs, *aggs, deg_in, deg_out, Wn, b.reshape(1, 512))


def _final(hs, aggs, deg_in, Ws, Wn, b, Wc, bc):
    def body(*refs):
        h_refs = refs[:4]
        a_refs = refs[4:8]
        di, ws_ref, wn_ref, b_ref, wc_ref, bc_ref = refs[8:14]
        out = refs[14]
        accr = refs[15]
        i = pl.program_id(0)
        h = jnp.concatenate([r[...] for r in h_refs], axis=1)
        agg = jnp.concatenate([r[...] for r in a_refs], axis=1)
        hn = agg * (1.0 / jnp.maximum(di[...], 1.0))
        o = (jnp.dot(h, ws_ref[...], preferred_element_type=jnp.float32)
             + jnp.dot(hn, wn_ref[...], preferred_element_type=jnp.float32)
             + b_ref[...])
        o = jnp.maximum(o, 0.0)
        part = jnp.sum(o, axis=0, keepdims=True)

        @pl.when(i == 0)
        def _():
            accr[...] = part

        @pl.when(i > 0)
        def _():
            accr[...] = accr[...] + part

        out[...] = (jnp.dot(accr[...] * (1.0 / N), wc_ref[...],
                            preferred_element_type=jnp.float32)
                    + bc_ref[...])

    return pl.pallas_call(
        body, grid=(NBLK,),
        in_specs=(_row_specs([(R, LANE)] * 8 + [(R, 1)])
                  + [_full_spec((512, 512)), _full_spec((512, 512)),
                     _full_spec((1, 512)), _full_spec((512, 40)),
                     _full_spec((1, 40))]),
        out_specs=pl.BlockSpec((1, 40), lambda i: (0, 0)),
        out_shape=jax.ShapeDtypeStruct((1, 40), jnp.float32),
        scratch_shapes=[pltpu.VMEM((1, 512), jnp.float32)],
    )(*hs, *aggs, deg_in, Ws, Wn, b.reshape(1, 512), Wc, bc.reshape(1, 40))


def kernel(x, edge_index, W1, b1, W2, b2, W3, b3, Ws1, Wn1, bS1, Wng, bg,
           Ws3, Wn3, bS3, Wc, bc):
    src = edge_index[0]
    dst = edge_index[1]

    deg_in_w, deg_out_w = _DEG(src, dst)
    deg_in = deg_in_w[:N, :1]
    deg_out = deg_out_w[:N, :1]

    m0 = _scale0(x, deg_out)
    a1 = _AGG2(m0[0], m0[1], src, dst)
    h1 = _gconv(a1, deg_in, W1, b1)

    a2 = _AGG4(*h1, src, dst)
    m2 = _sage_mean_scaled(h1, a2, deg_in, deg_out, Ws1, Wn1, bS1)

    a3 = _AGG4(*m2, src, dst)
    h3 = _gconv(a3, deg_in, W2, b2)

    a4 = _AGG4(*h3, src, dst)
    m4 = _sage_gcn_scaled(h3, a4, deg_in, deg_out, Wng, bg)

    a5 = _AGG4(*m4, src, dst)
    h5 = _gconv(a5, deg_in, W3, b3)

    a6 = _AGG4(*h5, src, dst)
    return _final(h5, a6, deg_in, Ws3, Wn3, bS3, Wc, bc)
